# baseline (speedup 1.0000x reference)
import functools

import jax
import jax.numpy as jnp
from jax import lax
from jax.experimental import pallas as pl
from jax.experimental.pallas import tpu as pltpu
from jax.experimental.pallas import tpu_sc as plsc

_E = 8
_TOP_P = 0.7
_EPS = 1e-10
_NC = 2
_NS = 16
_NW = _NC * _NS
_L = 16


def _make_probs_block(tiles_per_step, tpw):
    def body(x_ref, wg_ref, p_ref, ent_ref, ent_acc):
        i = pl.program_id(0)
        nsteps = pl.num_programs(0)

        @pl.when(i == 0)
        def _init():
            ent_acc[...] = jnp.zeros_like(ent_acc)

        x = x_ref[...]
        wg = wg_ref[...]
        logits = jax.lax.dot_general(
            x, wg, (((1,), (1,)), ((), ())),
            preferred_element_type=jnp.float32)
        lt = logits.T

        m = jnp.max(lt, axis=0, keepdims=True)
        ex = jnp.exp(lt - m)
        p = ex / jnp.sum(ex, axis=0, keepdims=True)
        for j in range(tiles_per_step):
            p_ref[j] = p[:, j * tpw:(j + 1) * tpw]
        ent_acc[...] += p * jnp.log(p + _EPS)

        @pl.when(i == nsteps - 1)
        def _fin():
            ent_ref[0, 0] = -jnp.sum(ent_acc[...])

    return body


def _sc_gate(p_hbm, kept_hbm, dec_hbm, imp_hbm, pv, kv, dv, iv):
    tpw = p_hbm.shape[2]
    wid = lax.axis_index("s") * _NC + lax.axis_index("c")
    pltpu.sync_copy(p_hbm.at[wid], pv)

    def chunk(c, acc):
        off = c * _L
        ps = [pv[e, pl.ds(off, _L)] for e in range(_E)]
        cnt = jnp.zeros((_L,), jnp.int32)
        acc = list(acc)
        for e in range(_E):
            s_above = jnp.zeros((_L,), jnp.float32)
            rank = jnp.zeros((_L,), jnp.int32)
            for j in range(_E):
                if j == e:
                    continue
                higher = (ps[j] >= ps[e]) if j < e else (ps[j] > ps[e])
                s_above += jnp.where(higher, ps[j], 0.0)
                rank += jnp.where(higher, 1, 0)
            kept = s_above <= _TOP_P
            kv[e, pl.ds(off, _L)] = jnp.where(kept, 1.0, 0.0)
            cnt += jnp.where(kept, 1, 0)
            contrib = jnp.where(kept, ps[e], 0.0)
            for k in range(_E):
                acc[k] = acc[k] + jnp.where(rank == k, contrib, 0.0)
        dv[pl.ds(off, _L)] = jnp.where(cnt > 1, 1, 0)
        return tuple(acc)

    acc0 = tuple(jnp.zeros((_L,), jnp.float32) for _ in range(_E))
    acc = lax.fori_loop(0, tpw // _L, chunk, acc0)
    for k in range(_E):
        iv[k, :] = acc[k]

    pltpu.sync_copy(kv, kept_hbm.at[wid])
    pltpu.sync_copy(dv, dec_hbm.at[pl.ds(wid * tpw, tpw)])
    pltpu.sync_copy(iv, imp_hbm.at[wid])


def _make_loss_combine(n_tokens):
    def body(imp_ref, ent_ref, loss_ref):
        parts = imp_ref[...]
        per_rank = jnp.sum(parts, axis=0)
        imp = jnp.sum(per_rank, axis=1, keepdims=True)
        mean = jnp.mean(imp)
        var = jnp.sum((imp - mean) ** 2) / (_E - 1)
        loss_imp = var / (mean * mean + _EPS)
        loss_ref[0, 0] = loss_imp + 0.1 * (ent_ref[0, 0] / n_tokens)
    return body


@functools.partial(jax.jit, static_argnames=())
def kernel(x, W_gate, W_noise):
    del W_noise
    b, n, d = x.shape
    e = W_gate.shape[0]
    bn = b * n
    t = 2048
    grid = bn // t
    tpw = bn // _NW
    tps = t // tpw
    x_flat = x.reshape(bn, d)

    p_t, ent = pl.pallas_call(
        _make_probs_block(tps, tpw),
        grid=(grid,),
        in_specs=[
            pl.BlockSpec((t, d), lambda i: (i, 0)),
            pl.BlockSpec((e, d), lambda i: (0, 0)),
        ],
        out_specs=[
            pl.BlockSpec((tps, e, tpw), lambda i: (i, 0, 0)),
            pl.BlockSpec(memory_space=pltpu.SMEM),
        ],
        out_shape=[
            jax.ShapeDtypeStruct((_NW, e, tpw), jnp.float32),
            jax.ShapeDtypeStruct((1, 1), jnp.float32),
        ],
        scratch_shapes=[
            pltpu.VMEM((e, t), jnp.float32),
        ],
        compiler_params=pltpu.CompilerParams(
            dimension_semantics=("arbitrary",),
        ),
    )(x_flat, W_gate)

    gate = pl.kernel(
        _sc_gate,
        out_type=[
            jax.ShapeDtypeStruct((_NW, e, tpw), jnp.float32),
            jax.ShapeDtypeStruct((bn,), jnp.int32),
            jax.ShapeDtypeStruct((_NW, e, _L), jnp.float32),
        ],
        mesh=plsc.VectorSubcoreMesh(
            core_axis_name="c", subcore_axis_name="s",
            num_cores=_NC, num_subcores=_NS),
        scratch_types=[
            pltpu.VMEM((e, tpw), jnp.float32),
            pltpu.VMEM((e, tpw), jnp.float32),
            pltpu.VMEM((tpw,), jnp.int32),
            pltpu.VMEM((e, _L), jnp.float32),
        ],
    )
    kept_t, dec, imp_parts = gate(p_t)

    loss = pl.pallas_call(
        _make_loss_combine(bn),
        in_specs=[
            pl.BlockSpec(memory_space=pltpu.MemorySpace.VMEM),
            pl.BlockSpec(memory_space=pltpu.SMEM),
        ],
        out_specs=pl.BlockSpec(memory_space=pltpu.SMEM),
        out_shape=jax.ShapeDtypeStruct((1, 1), jnp.float32),
    )(imp_parts, ent)

    expert_weights = (
        kept_t.transpose(0, 2, 1).reshape(bn, e).astype(jnp.bool_)
        .reshape(b, n, e))
    expert_decisions = dec.reshape(b, n)
    gating_loss = loss.reshape(())
    return expert_weights, expert_decisions, gating_loss


# speedup vs baseline: 1.0502x; 1.0502x over previous
import functools

import jax
import jax.numpy as jnp
from jax import lax
from jax.experimental import pallas as pl
from jax.experimental.pallas import tpu as pltpu
from jax.experimental.pallas import tpu_sc as plsc

_E = 8
_TOP_P = 0.7
_EPS = 1e-10
_NC = 2
_NS = 16
_NW = _NC * _NS
_L = 16


def _make_probs_block(tiles_per_step, tpw):
    def body(x_ref, wg_ref, p_ref, ent_ref, ent_acc):
        i = pl.program_id(0)
        nsteps = pl.num_programs(0)

        @pl.when(i == 0)
        def _init():
            ent_acc[...] = jnp.zeros_like(ent_acc)

        x = x_ref[...]
        wg = wg_ref[...]
        logits = jax.lax.dot_general(
            x, wg, (((1,), (1,)), ((), ())),
            preferred_element_type=jnp.float32)
        lt = logits.T

        m = jnp.max(lt, axis=0, keepdims=True)
        ex = jnp.exp(lt - m)
        p = ex / jnp.sum(ex, axis=0, keepdims=True)
        for j in range(tiles_per_step):
            p_ref[j] = p[:, j * tpw:(j + 1) * tpw]
        ent_acc[...] += p * jnp.log(p + _EPS)

        @pl.when(i == nsteps - 1)
        def _fin():
            ent_ref[0, 0] = -jnp.sum(ent_acc[...])

    return body


def _sc_gate(p_hbm, kept_hbm, dec_hbm, imp_hbm, pv, kv, dv, iv):
    tpw = p_hbm.shape[2]
    wid = lax.axis_index("s") * _NC + lax.axis_index("c")
    pltpu.sync_copy(p_hbm.at[wid], pv)

    def chunk(c, acc):
        off = c * _L
        ps = [pv[e, pl.ds(off, _L)] for e in range(_E)]
        cnt = jnp.zeros((_L,), jnp.int32)
        acc = list(acc)
        for e in range(_E):
            s_above = jnp.zeros((_L,), jnp.float32)
            rank = jnp.zeros((_L,), jnp.int32)
            for j in range(_E):
                if j == e:
                    continue
                higher = (ps[j] >= ps[e]) if j < e else (ps[j] > ps[e])
                s_above += jnp.where(higher, ps[j], 0.0)
                rank += jnp.where(higher, 1, 0)
            kept = s_above <= _TOP_P
            kv[e, pl.ds(off, _L)] = jnp.where(kept, 1.0, 0.0)
            cnt += jnp.where(kept, 1, 0)
            contrib = jnp.where(kept, ps[e], 0.0)
            for k in range(_E):
                acc[k] = acc[k] + jnp.where(rank == k, contrib, 0.0)
        dv[pl.ds(off, _L)] = jnp.where(cnt > 1, 1, 0)
        return tuple(acc)

    acc0 = tuple(jnp.zeros((_L,), jnp.float32) for _ in range(_E))
    acc = lax.fori_loop(0, tpw // _L, chunk, acc0)
    for k in range(_E):
        iv[k, :] = acc[k]

    pltpu.sync_copy(kv, kept_hbm.at[wid])
    pltpu.sync_copy(dv, dec_hbm.at[pl.ds(wid * tpw, tpw)])
    pltpu.sync_copy(iv, imp_hbm.at[wid])


def _make_loss_combine(n_tokens):
    def body(imp_ref, ent_ref, loss_ref):
        parts = imp_ref[...]
        per_rank = jnp.sum(parts, axis=0)
        imp = jnp.sum(per_rank, axis=1, keepdims=True)
        mean = jnp.mean(imp)
        var = jnp.sum((imp - mean) ** 2) / (_E - 1)
        loss_imp = var / (mean * mean + _EPS)
        loss_ref[0, 0] = loss_imp + 0.1 * (ent_ref[0, 0] / n_tokens)
    return body


@functools.partial(jax.jit, static_argnames=())
def kernel(x, W_gate, W_noise):
    del W_noise
    b, n, d = x.shape
    e = W_gate.shape[0]
    bn = b * n
    t = 1024
    grid = bn // t
    tpw = bn // _NW
    tps = t // tpw
    x_flat = x.reshape(bn, d)

    p_t, ent = pl.pallas_call(
        _make_probs_block(tps, tpw),
        grid=(grid,),
        in_specs=[
            pl.BlockSpec((t, d), lambda i: (i, 0)),
            pl.BlockSpec((e, d), lambda i: (0, 0)),
        ],
        out_specs=[
            pl.BlockSpec((tps, e, tpw), lambda i: (i, 0, 0)),
            pl.BlockSpec(memory_space=pltpu.SMEM),
        ],
        out_shape=[
            jax.ShapeDtypeStruct((_NW, e, tpw), jnp.float32),
            jax.ShapeDtypeStruct((1, 1), jnp.float32),
        ],
        scratch_shapes=[
            pltpu.VMEM((e, t), jnp.float32),
        ],
        compiler_params=pltpu.CompilerParams(
            dimension_semantics=("arbitrary",),
        ),
    )(x_flat, W_gate)

    gate = pl.kernel(
        _sc_gate,
        out_type=[
            jax.ShapeDtypeStruct((_NW, e, tpw), jnp.float32),
            jax.ShapeDtypeStruct((bn,), jnp.int32),
            jax.ShapeDtypeStruct((_NW, e, _L), jnp.float32),
        ],
        mesh=plsc.VectorSubcoreMesh(
            core_axis_name="c", subcore_axis_name="s",
            num_cores=_NC, num_subcores=_NS),
        scratch_types=[
            pltpu.VMEM((e, tpw), jnp.float32),
            pltpu.VMEM((e, tpw), jnp.float32),
            pltpu.VMEM((tpw,), jnp.int32),
            pltpu.VMEM((e, _L), jnp.float32),
        ],
    )
    kept_t, dec, imp_parts = gate(p_t)

    loss = pl.pallas_call(
        _make_loss_combine(bn),
        in_specs=[
            pl.BlockSpec(memory_space=pltpu.MemorySpace.VMEM),
            pl.BlockSpec(memory_space=pltpu.SMEM),
        ],
        out_specs=pl.BlockSpec(memory_space=pltpu.SMEM),
        out_shape=jax.ShapeDtypeStruct((1, 1), jnp.float32),
    )(imp_parts, ent)

    expert_weights = (
        kept_t.transpose(0, 2, 1).reshape(bn, e).astype(jnp.bool_)
        .reshape(b, n, e))
    expert_decisions = dec.reshape(b, n)
    gating_loss = loss.reshape(())
    return expert_weights, expert_decisions, gating_loss
